# baseline (speedup 1.0000x reference)
import functools
import math

import jax
import jax.numpy as jnp
from jax import lax
from jax.experimental import pallas as pl
from jax.experimental.pallas import tpu as pltpu
from jax.experimental.pallas import tpu_sc as plsc

N_WORKERS = 32
BLK = 8000
NEG_2PI2 = -2.0 * math.pi ** 2


@functools.partial(jax.jit, static_argnums=())
def _scaler_call(fcalc, s_flat, bins, table_pad, u_pad):
    n = fcalc.shape[0]
    assert n % BLK == 0
    nblocks = n // BLK
    blocks_per_worker = -(-nblocks // N_WORKERS)

    mesh = plsc.VectorSubcoreMesh(core_axis_name="c", subcore_axis_name="s")

    @functools.partial(
        pl.kernel,
        mesh=mesh,
        compiler_params=pltpu.CompilerParams(needs_layout_passes=False),
        out_type=jax.ShapeDtypeStruct((n,), jnp.float32),
        scratch_types=[
            pltpu.VMEM((BLK,), jnp.float32),
            pltpu.VMEM((BLK,), jnp.float32),
            pltpu.VMEM((BLK,), jnp.int32),
            pltpu.VMEM((BLK,), jnp.int32),
            pltpu.VMEM((3 * BLK,), jnp.float32),
            pltpu.VMEM((3 * BLK,), jnp.float32),
            pltpu.VMEM((BLK,), jnp.float32),
            pltpu.VMEM((BLK,), jnp.float32),
            pltpu.VMEM((32,), jnp.float32),
            pltpu.VMEM((96,), jnp.float32),
            pltpu.SemaphoreType.DMA,
            pltpu.SemaphoreType.DMA,
            pltpu.SemaphoreType.DMA,
            pltpu.SemaphoreType.DMA,
        ],
    )
    def scaler_kernel(fcalc_hbm, s_hbm, bins_hbm, table_hbm, coef_hbm, out_hbm,
                      f0, f1, b0, b1, s0, s1, o0, o1, tab_v, coef_v,
                      isem0, isem1, osem0, osem1):
        wid = lax.axis_index("s") * 2 + lax.axis_index("c")

        pltpu.sync_copy(table_hbm, tab_v)
        pltpu.sync_copy(coef_hbm, coef_v)
        tab_v[pl.ds(0, 16)] = jnp.exp(tab_v[pl.ds(0, 16)])
        tab_v[pl.ds(16, 16)] = jnp.exp(tab_v[pl.ds(16, 16)])

        a_xx = coef_v[pl.ds(0, 16)]
        a_yy = coef_v[pl.ds(16, 16)]
        a_zz = coef_v[pl.ds(32, 16)]
        a_xy = coef_v[pl.ds(48, 16)]
        a_xz = coef_v[pl.ds(64, 16)]
        a_yz = coef_v[pl.ds(80, 16)]

        lane3 = lax.iota(jnp.int32, 16) * 3

        bufs = [(f0, b0, s0, o0, isem0, osem0),
                (f1, b1, s1, o1, isem1, osem1)]

        def block_base(j):
            b = wid + N_WORKERS * j
            if (j + 1) * N_WORKERS > nblocks:
                b = jnp.where(b < nblocks, b, b - N_WORKERS)
            return b * BLK

        def start_in(j):
            base = block_base(j)
            f_v, bi_v, s_v, _, isem, _ = bufs[j % 2]
            return (
                pltpu.async_copy(fcalc_hbm.at[pl.ds(base, BLK)], f_v, isem),
                pltpu.async_copy(bins_hbm.at[pl.ds(base, BLK)], bi_v, isem),
                pltpu.async_copy(s_hbm.at[pl.ds(base * 3, 3 * BLK)], s_v, isem),
            )

        def start_out(j):
            base = block_base(j)
            _, _, _, o_v, _, osem = bufs[j % 2]
            return pltpu.async_copy(o_v, out_hbm.at[pl.ds(base, BLK)], osem)

        def compute(j):
            f_v, bi_v, s_v, o_v, _, _ = bufs[j % 2]

            def body(i, carry):
                off = i * 16
                xi = i * 48 + lane3
                x = plsc.load_gather(s_v, [xi])
                y = plsc.load_gather(s_v, [xi + 1])
                z = plsc.load_gather(s_v, [xi + 2])
                scale = plsc.load_gather(tab_v, [bi_v[pl.ds(off, 16)]])
                f = f_v[pl.ds(off, 16)]
                t0 = a_xx * x + a_xy * y + a_xz * z
                t1 = a_yy * y + a_yz * z
                t2 = a_zz * z
                q = x * t0 + y * t1 + z * t2
                o_v[pl.ds(off, 16)] = f * scale * jnp.exp(q)
                return carry

            lax.fori_loop(0, BLK // 16, body, 0)

        in_flight = {0: start_in(0)}
        out_flight = {}
        for j in range(blocks_per_worker):
            if j + 1 < blocks_per_worker:
                in_flight[j + 1] = start_in(j + 1)
            for c in in_flight.pop(j):
                c.wait()
            if j - 2 in out_flight:
                out_flight.pop(j - 2).wait()
            compute(j)
            out_flight[j] = start_out(j)
        for c in out_flight.values():
            c.wait()

    return scaler_kernel(fcalc, s_flat, bins, table_pad, u_pad)


def kernel(fcalc, s, bins, log_scale, U):
    s_flat = s.reshape(-1)
    table_pad = jnp.pad(log_scale, (0, 32 - log_scale.shape[0]))
    scal = jnp.stack([U[0], U[1], U[2],
                      2.0 * U[3], 2.0 * U[4], 2.0 * U[5]]) * NEG_2PI2
    coefs = jnp.repeat(scal, 16)
    return _scaler_call(fcalc, s_flat, bins.astype(jnp.int32), table_pad, coefs)


# speedup vs baseline: 63.0602x; 63.0602x over previous
import functools
import math

import jax
import jax.numpy as jnp
from jax import lax
from jax.experimental import pallas as pl
from jax.experimental.pallas import tpu as pltpu
from jax.experimental.pallas import tpu_sc as plsc

N_WORKERS = 32
BLK = 3200
NEG_2PI2 = -2.0 * math.pi ** 2


@jax.jit
def _scaler_call(fcalc, s_t, bins, table_pad, coefs):
    n = fcalc.shape[0]
    assert n % BLK == 0
    nblocks = n // BLK
    blocks_per_worker = -(-nblocks // N_WORKERS)

    mesh = plsc.VectorSubcoreMesh(core_axis_name="c", subcore_axis_name="s")

    @functools.partial(
        pl.kernel,
        mesh=mesh,
        compiler_params=pltpu.CompilerParams(needs_layout_passes=False),
        out_type=jax.ShapeDtypeStruct((n,), jnp.float32),
        scratch_types=[
            pltpu.VMEM((BLK,), jnp.float32),
            pltpu.VMEM((BLK,), jnp.float32),
            pltpu.VMEM((BLK,), jnp.int32),
            pltpu.VMEM((BLK,), jnp.int32),
            pltpu.VMEM((3, BLK), jnp.float32),
            pltpu.VMEM((3, BLK), jnp.float32),
            pltpu.VMEM((BLK,), jnp.float32),
            pltpu.VMEM((BLK,), jnp.float32),
            pltpu.VMEM((32,), jnp.float32),
            pltpu.VMEM((96,), jnp.float32),
            pltpu.SemaphoreType.DMA,
            pltpu.SemaphoreType.DMA,
            pltpu.SemaphoreType.DMA,
            pltpu.SemaphoreType.DMA,
        ],
    )
    def scaler_kernel(fcalc_hbm, st_hbm, bins_hbm, table_hbm, coef_hbm, out_hbm,
                      f0, f1, b0, b1, s0, s1, o0, o1,
                      tab_v, coef_v, isem0, isem1, osem0, osem1):
        wid = lax.axis_index("s") * 2 + lax.axis_index("c")

        pltpu.sync_copy(table_hbm, tab_v)
        pltpu.sync_copy(coef_hbm, coef_v)
        tab_v[pl.ds(0, 16)] = jnp.exp(tab_v[pl.ds(0, 16)])
        tab_v[pl.ds(16, 16)] = jnp.exp(tab_v[pl.ds(16, 16)])

        a_xx = coef_v[pl.ds(0, 16)]
        a_yy = coef_v[pl.ds(16, 16)]
        a_zz = coef_v[pl.ds(32, 16)]
        a_xy = coef_v[pl.ds(48, 16)]
        a_xz = coef_v[pl.ds(64, 16)]
        a_yz = coef_v[pl.ds(80, 16)]

        bufs = [(f0, b0, s0, o0, isem0, osem0),
                (f1, b1, s1, o1, isem1, osem1)]

        def block_base(j):
            b = wid + N_WORKERS * j
            if (j + 1) * N_WORKERS > nblocks:
                b = jnp.where(b < nblocks, b, b - N_WORKERS)
            return b * BLK

        def start_in(j):
            base = block_base(j)
            f_v, bi_v, s_v, _, isem, _ = bufs[j % 2]
            return (
                pltpu.async_copy(fcalc_hbm.at[pl.ds(base, BLK)], f_v, isem),
                pltpu.async_copy(bins_hbm.at[pl.ds(base, BLK)], bi_v, isem),
                pltpu.async_copy(st_hbm.at[:, pl.ds(base, BLK)], s_v, isem),
            )

        def start_out(j):
            base = block_base(j)
            o_v, osem = bufs[j % 2][3], bufs[j % 2][5]
            return pltpu.async_copy(o_v, out_hbm.at[pl.ds(base, BLK)], osem)

        def compute(j):
            f_v, bi_v, s_v, o_v = bufs[j % 2][:4]

            def body(i, carry):
                off = i * 16
                x = s_v[0, pl.ds(off, 16)]
                y = s_v[1, pl.ds(off, 16)]
                z = s_v[2, pl.ds(off, 16)]
                scale = plsc.load_gather(tab_v, [bi_v[pl.ds(off, 16)]])
                f = f_v[pl.ds(off, 16)]
                t0 = a_xx * x + a_xy * y + a_xz * z
                t1 = a_yy * y + a_yz * z
                t2 = a_zz * z
                q = x * t0 + y * t1 + z * t2
                o_v[pl.ds(off, 16)] = f * scale * jnp.exp(q)
                return carry

            lax.fori_loop(0, BLK // 16, body, 0)

        in_flight = {0: start_in(0)}
        out_flight = {}
        for j in range(blocks_per_worker):
            if j + 1 < blocks_per_worker:
                in_flight[j + 1] = start_in(j + 1)
            for c in in_flight.pop(j):
                c.wait()
            if j - 2 in out_flight:
                out_flight.pop(j - 2).wait()
            compute(j)
            out_flight[j] = start_out(j)
        for c in out_flight.values():
            c.wait()

    return scaler_kernel(fcalc, s_t, bins, table_pad, coefs)


def kernel(fcalc, s, bins, log_scale, U):
    s_t = s.T
    table_pad = jnp.pad(log_scale, (0, 32 - log_scale.shape[0]))
    scal = jnp.stack([U[0], U[1], U[2],
                      2.0 * U[3], 2.0 * U[4], 2.0 * U[5]]) * NEG_2PI2
    coefs = jnp.repeat(scal, 16)
    return _scaler_call(fcalc, s_t, bins.astype(jnp.int32), table_pad, coefs)


# speedup vs baseline: 123.0055x; 1.9506x over previous
import functools
import math

import jax
import jax.numpy as jnp
from jax import lax
from jax.experimental import pallas as pl
from jax.experimental.pallas import tpu as pltpu
from jax.experimental.pallas import tpu_sc as plsc

N_WORKERS = 32
BLK = 3200
NEG_2PI2 = -2.0 * math.pi ** 2


@jax.jit
def _scaler_call(fcalc, s_t, bins, table_pad, coefs):
    n = fcalc.shape[0]
    assert n % BLK == 0
    nblocks = n // BLK
    blocks_per_worker = -(-nblocks // N_WORKERS)

    mesh = plsc.VectorSubcoreMesh(core_axis_name="c", subcore_axis_name="s")

    @functools.partial(
        pl.kernel,
        mesh=mesh,
        compiler_params=pltpu.CompilerParams(needs_layout_passes=False),
        out_type=jax.ShapeDtypeStruct((n,), jnp.float32),
        scratch_types=[
            pltpu.VMEM((BLK,), jnp.float32),
            pltpu.VMEM((BLK,), jnp.float32),
            pltpu.VMEM((BLK,), jnp.int32),
            pltpu.VMEM((BLK,), jnp.int32),
            pltpu.VMEM((3, BLK), jnp.float32),
            pltpu.VMEM((3, BLK), jnp.float32),
            pltpu.VMEM((BLK,), jnp.float32),
            pltpu.VMEM((BLK,), jnp.float32),
            pltpu.VMEM((32,), jnp.float32),
            pltpu.VMEM((96,), jnp.float32),
            pltpu.SemaphoreType.DMA,
            pltpu.SemaphoreType.DMA,
            pltpu.SemaphoreType.DMA,
            pltpu.SemaphoreType.DMA,
        ],
    )
    def scaler_kernel(fcalc_hbm, st_hbm, bins_hbm, table_hbm, coef_hbm, out_hbm,
                      f0, f1, b0, b1, s0, s1, o0, o1,
                      tab_v, coef_v, isem0, isem1, osem0, osem1):
        wid = lax.axis_index("s") * 2 + lax.axis_index("c")

        pltpu.sync_copy(table_hbm, tab_v)
        pltpu.sync_copy(coef_hbm, coef_v)
        tab_v[pl.ds(0, 16)] = jnp.exp(tab_v[pl.ds(0, 16)])
        tab_v[pl.ds(16, 16)] = jnp.exp(tab_v[pl.ds(16, 16)])

        a_xx = coef_v[pl.ds(0, 16)]
        a_yy = coef_v[pl.ds(16, 16)]
        a_zz = coef_v[pl.ds(32, 16)]
        a_xy = coef_v[pl.ds(48, 16)]
        a_xz = coef_v[pl.ds(64, 16)]
        a_yz = coef_v[pl.ds(80, 16)]

        bufs = [(f0, b0, s0, o0, isem0, osem0),
                (f1, b1, s1, o1, isem1, osem1)]

        def block_base(j):
            b = wid + N_WORKERS * j
            if (j + 1) * N_WORKERS > nblocks:
                b = jnp.where(b < nblocks, b, b - N_WORKERS)
            return b * BLK

        def start_in(j):
            base = block_base(j)
            f_v, bi_v, s_v, _, isem, _ = bufs[j % 2]
            return (
                pltpu.async_copy(fcalc_hbm.at[pl.ds(base, BLK)], f_v, isem),
                pltpu.async_copy(bins_hbm.at[pl.ds(base, BLK)], bi_v, isem),
                pltpu.async_copy(st_hbm.at[:, pl.ds(base, BLK)], s_v, isem),
            )

        def start_out(j):
            base = block_base(j)
            o_v, osem = bufs[j % 2][3], bufs[j % 2][5]
            return pltpu.async_copy(o_v, out_hbm.at[pl.ds(base, BLK)], osem)

        def compute(j):
            f_v, bi_v, s_v, o_v = bufs[j % 2][:4]

            @plsc.parallel_loop(0, BLK, 16, unroll=4)
            def body(off):
                x = s_v[0, pl.ds(off, 16)]
                y = s_v[1, pl.ds(off, 16)]
                z = s_v[2, pl.ds(off, 16)]
                scale = plsc.load_gather(tab_v, [bi_v[pl.ds(off, 16)]])
                f = f_v[pl.ds(off, 16)]
                t0 = a_xx * x + a_xy * y + a_xz * z
                t1 = a_yy * y + a_yz * z
                t2 = a_zz * z
                q = x * t0 + y * t1 + z * t2
                o_v[pl.ds(off, 16)] = f * scale * jnp.exp(q)

        in_flight = {0: start_in(0)}
        out_flight = {}
        for j in range(blocks_per_worker):
            if j + 1 < blocks_per_worker:
                in_flight[j + 1] = start_in(j + 1)
            for c in in_flight.pop(j):
                c.wait()
            if j - 2 in out_flight:
                out_flight.pop(j - 2).wait()
            compute(j)
            out_flight[j] = start_out(j)
        for c in out_flight.values():
            c.wait()

    return scaler_kernel(fcalc, s_t, bins, table_pad, coefs)


def kernel(fcalc, s, bins, log_scale, U):
    s_t = s.T
    table_pad = jnp.pad(log_scale, (0, 32 - log_scale.shape[0]))
    scal = jnp.stack([U[0], U[1], U[2],
                      2.0 * U[3], 2.0 * U[4], 2.0 * U[5]]) * NEG_2PI2
    coefs = jnp.repeat(scal, 16)
    return _scaler_call(fcalc, s_t, bins.astype(jnp.int32), table_pad, coefs)
